# baseline (speedup 1.0000x reference)
import functools

import jax
import jax.numpy as jnp
from jax import lax
from jax.experimental import pallas as pl
from jax.experimental.pallas import tpu as pltpu
from jax.experimental.pallas import tpu_sc as plsc

_NUM_BUCKETS = 100000
_DIM = 100
_DIMP = 128
_PRIME0, _PRIME1 = 31, 1009
_OFF0, _OFF1 = 7, 433
_BATCH = 16384
_OUT = 300

_NC = 2
_NS = 16
_NW = _NC * _NS
_BPW = _BATCH // _NW
_SUB = 256
_LANES = 16

_sc_mesh = plsc.VectorSubcoreMesh(core_axis_name="c", subcore_axis_name="s")


def _sc_gather_tables(x, table_p):
    @functools.partial(
        pl.kernel,
        mesh=_sc_mesh,
        out_type=(
            jax.ShapeDtypeStruct((_BATCH, _DIMP), jnp.float32),
            jax.ShapeDtypeStruct((_BATCH, _DIMP), jnp.float32),
        ),
        scratch_types=[
            pltpu.VMEM((_BPW,), jnp.int32),
            pltpu.VMEM((_SUB,), jnp.int32),
            pltpu.VMEM((_SUB,), jnp.int32),
            pltpu.VMEM((_SUB, _DIMP), jnp.float32),
            pltpu.VMEM((_SUB, _DIMP), jnp.float32),
            pltpu.SemaphoreType.DMA,
        ],
    )
    def tab_kernel(x_hbm, table_hbm, t0_hbm, t1_hbm,
                   x_v, h0_v, h1_v, t0_v, t1_v, sem):
        wid = lax.axis_index("s") * _NC + lax.axis_index("c")
        base = wid * _BPW
        pltpu.sync_copy(x_hbm.at[pl.ds(base, _BPW)], x_v)

        for s in range(_BPW // _SUB):
            def hash_body(i, carry, s=s):
                xx = x_v[pl.ds(s * _SUB + i * _LANES, _LANES)]
                sl = pl.ds(i * _LANES, _LANES)
                h0_v[sl] = (xx * _PRIME0 + _OFF0) % _NUM_BUCKETS
                h1_v[sl] = (xx * _PRIME1 + _OFF1) % _NUM_BUCKETS
                return carry

            lax.fori_loop(0, _SUB // _LANES, hash_body, 0)

            cp0 = pltpu.async_copy(table_hbm.at[h0_v], t0_v, sem)
            cp1 = pltpu.async_copy(table_hbm.at[h1_v], t1_v, sem)
            cp0.wait()
            cp1.wait()

            out_sl = pl.ds(base + s * _SUB, _SUB)
            pltpu.sync_copy(t0_v, t0_hbm.at[out_sl])
            pltpu.sync_copy(t1_v, t1_hbm.at[out_sl])

    return tab_kernel(x, table_p)


def _sc_gather_imp(x, imp0, imp1):
    @functools.partial(
        pl.kernel,
        mesh=_sc_mesh,
        out_type=(
            jax.ShapeDtypeStruct((_BATCH,), jnp.float32),
            jax.ShapeDtypeStruct((_BATCH,), jnp.float32),
        ),
        scratch_types=[
            pltpu.VMEM((_BPW,), jnp.int32),
            pltpu.VMEM((_BPW,), jnp.float32),
            pltpu.VMEM((_BPW,), jnp.float32),
            pltpu.SemaphoreType.DMA,
        ],
    )
    def imp_kernel(x_hbm, imp0_hbm, imp1_hbm, w0_hbm, w1_hbm,
                   x_v, i0_v, i1_v, sem):
        wid = lax.axis_index("s") * _NC + lax.axis_index("c")
        base = wid * _BPW
        sl = pl.ds(base, _BPW)
        pltpu.sync_copy(x_hbm.at[sl], x_v)
        cp0 = pltpu.async_copy(imp0_hbm.at[x_v], i0_v, sem)
        cp1 = pltpu.async_copy(imp1_hbm.at[x_v], i1_v, sem)
        cp0.wait()
        cp1.wait()
        pltpu.sync_copy(i0_v, w0_hbm.at[sl])
        pltpu.sync_copy(i1_v, w1_hbm.at[sl])

    return imp_kernel(x, imp0, imp1)


_PAD_BLK = 8192


def _pad_body(t_ref, o_ref):
    o_ref[:, : _DIM] = t_ref[...].T
    o_ref[:, _DIM:] = jnp.zeros((_PAD_BLK, _DIMP - _DIM), jnp.float32)


def _pad_table(table_t):
    nblk = (_NUM_BUCKETS + _PAD_BLK - 1) // _PAD_BLK
    return pl.pallas_call(
        _pad_body,
        grid=(nblk,),
        in_specs=[pl.BlockSpec((_DIM, _PAD_BLK), lambda i: (0, i))],
        out_specs=pl.BlockSpec((_PAD_BLK, _DIMP), lambda i: (i, 0)),
        out_shape=jax.ShapeDtypeStruct((_NUM_BUCKETS, _DIMP), jnp.float32),
    )(table_t)


_TC_BLK = 4096


def _tc_head_body(t0_ref, t1_ref, iw_ref, w_ref, b_ref, o_ref):
    t0 = t0_ref[...]
    t1 = t1_ref[...]
    iw = iw_ref[...]
    w0 = iw[:, 0:1]
    w1 = iw[:, 1:2]
    emb = w0 * t0 + w1 * t1
    out = lax.dot_general(w_ref[...], emb, (((1,), (1,)), ((), ())),
                          preferred_element_type=jnp.float32)
    out = out + b_ref[...]
    m = jnp.max(out, axis=0, keepdims=True)
    e = jnp.exp(out - m)
    s = jnp.sum(e, axis=0, keepdims=True)
    o_ref[...] = out - m - jnp.log(s)


def _tc_head(t0p, t1p, iw, fc_wp, fc_b2):
    nblk = _BATCH // _TC_BLK
    return pl.pallas_call(
        _tc_head_body,
        grid=(nblk,),
        in_specs=[
            pl.BlockSpec((_TC_BLK, _DIMP), lambda i: (i, 0)),
            pl.BlockSpec((_TC_BLK, _DIMP), lambda i: (i, 0)),
            pl.BlockSpec((_TC_BLK, 2), lambda i: (i, 0)),
            pl.BlockSpec((_OUT, _DIMP), lambda i: (0, 0)),
            pl.BlockSpec((_OUT, 1), lambda i: (0, 0)),
        ],
        out_specs=pl.BlockSpec((_OUT, _TC_BLK), lambda i: (0, i)),
        out_shape=jax.ShapeDtypeStruct((_OUT, _BATCH), jnp.float32),
    )(t0p, t1p, iw, fc_wp, fc_b2)


def kernel(x, table, importance, fc_w, fc_b):
    table_p = _pad_table(table.T)
    table_p, imp_b = lax.optimization_barrier((table_p, importance))
    imp0 = imp_b @ jnp.array([1.0, 0.0], jnp.float32)
    imp1 = imp_b @ jnp.array([0.0, 1.0], jnp.float32)
    fc_wpt = jnp.pad(fc_w.T, ((0, 0), (0, _DIMP - _DIM)))
    t0p, t1p = _sc_gather_tables(x, table_p)
    imp0, imp1 = lax.optimization_barrier((imp0, imp1, t0p))[:2]
    w0, w1 = _sc_gather_imp(x, imp0, imp1)
    iw = jnp.stack([w0, w1], axis=1)
    out_t = _tc_head(t0p, t1p, iw, fc_wpt, fc_b.reshape(_OUT, 1))
    return out_t.T


# speedup vs baseline: 1.3394x; 1.3394x over previous
import functools

import jax
import jax.numpy as jnp
from jax import lax
from jax.experimental import pallas as pl
from jax.experimental.pallas import tpu as pltpu
from jax.experimental.pallas import tpu_sc as plsc

_NUM_BUCKETS = 100000
_DIM = 100
_DIMP = 128
_PRIME0, _PRIME1 = 31, 1009
_OFF0, _OFF1 = 7, 433
_BATCH = 16384
_OUT = 300

_NC = 2
_NS = 16
_NW = _NC * _NS
_BPW = _BATCH // _NW
_SUB = 256
_LANES = 16

_sc_mesh = plsc.VectorSubcoreMesh(core_axis_name="c", subcore_axis_name="s")


def _sc_gather_tables(x, table_p):
    @functools.partial(
        pl.kernel,
        mesh=_sc_mesh,
        out_type=(
            jax.ShapeDtypeStruct((_BATCH, _DIMP), jnp.float32),
            jax.ShapeDtypeStruct((_BATCH, _DIMP), jnp.float32),
        ),
        scratch_types=[
            pltpu.VMEM((_BPW,), jnp.int32),
            pltpu.VMEM((_SUB,), jnp.int32),
            pltpu.VMEM((_SUB,), jnp.int32),
            pltpu.VMEM((_SUB, _DIMP), jnp.float32),
            pltpu.VMEM((_SUB, _DIMP), jnp.float32),
            pltpu.SemaphoreType.DMA,
        ],
    )
    def tab_kernel(x_hbm, table_hbm, t0_hbm, t1_hbm,
                   x_v, h0_v, h1_v, t0_v, t1_v, sem):
        wid = lax.axis_index("s") * _NC + lax.axis_index("c")
        base = wid * _BPW
        pltpu.sync_copy(x_hbm.at[pl.ds(base, _BPW)], x_v)

        for s in range(_BPW // _SUB):
            def hash_body(i, carry, s=s):
                xx = x_v[pl.ds(s * _SUB + i * _LANES, _LANES)]
                sl = pl.ds(i * _LANES, _LANES)
                h0_v[sl] = (xx * _PRIME0 + _OFF0) % _NUM_BUCKETS
                h1_v[sl] = (xx * _PRIME1 + _OFF1) % _NUM_BUCKETS
                return carry

            lax.fori_loop(0, _SUB // _LANES, hash_body, 0)

            cp0 = pltpu.async_copy(table_hbm.at[h0_v], t0_v, sem)
            cp1 = pltpu.async_copy(table_hbm.at[h1_v], t1_v, sem)
            cp0.wait()
            cp1.wait()

            out_sl = pl.ds(base + s * _SUB, _SUB)
            pltpu.sync_copy(t0_v, t0_hbm.at[out_sl])
            pltpu.sync_copy(t1_v, t1_hbm.at[out_sl])

    return tab_kernel(x, table_p)


def _sc_gather_imp(x, imp0, imp1):
    @functools.partial(
        pl.kernel,
        mesh=_sc_mesh,
        out_type=(
            jax.ShapeDtypeStruct((_BATCH,), jnp.float32),
            jax.ShapeDtypeStruct((_BATCH,), jnp.float32),
        ),
        scratch_types=[
            pltpu.VMEM((_BPW,), jnp.int32),
            pltpu.VMEM((_BPW,), jnp.float32),
            pltpu.VMEM((_BPW,), jnp.float32),
            pltpu.SemaphoreType.DMA,
        ],
    )
    def imp_kernel(x_hbm, imp0_hbm, imp1_hbm, w0_hbm, w1_hbm,
                   x_v, i0_v, i1_v, sem):
        wid = lax.axis_index("s") * _NC + lax.axis_index("c")
        base = wid * _BPW
        sl = pl.ds(base, _BPW)
        pltpu.sync_copy(x_hbm.at[sl], x_v)
        cp0 = pltpu.async_copy(imp0_hbm.at[x_v], i0_v, sem)
        cp1 = pltpu.async_copy(imp1_hbm.at[x_v], i1_v, sem)
        cp0.wait()
        cp1.wait()
        pltpu.sync_copy(i0_v, w0_hbm.at[sl])
        pltpu.sync_copy(i1_v, w1_hbm.at[sl])

    return imp_kernel(x, imp0, imp1)


_PAD_BLK = 8192


def _pad_body(t_ref, o_ref):
    o_ref[:, : _DIM] = t_ref[...].T
    o_ref[:, _DIM:] = jnp.zeros((_PAD_BLK, _DIMP - _DIM), jnp.float32)


def _pad_table(table_t):
    nblk = (_NUM_BUCKETS + _PAD_BLK - 1) // _PAD_BLK
    return pl.pallas_call(
        _pad_body,
        grid=(nblk,),
        in_specs=[pl.BlockSpec((_DIM, _PAD_BLK), lambda i: (0, i))],
        out_specs=pl.BlockSpec((_PAD_BLK, _DIMP), lambda i: (i, 0)),
        out_shape=jax.ShapeDtypeStruct((_NUM_BUCKETS, _DIMP), jnp.float32),
    )(table_t)


_TC_BLK = 4096


def _tc_head_body(t0_ref, t1_ref, iw_ref, w_ref, b_ref, o_ref):
    t0 = t0_ref[...]
    t1 = t1_ref[...]
    iw = iw_ref[...]
    w0 = iw[:, 0:1]
    w1 = iw[:, 1:2]
    emb = w0 * t0 + w1 * t1
    out = lax.dot_general(w_ref[...], emb, (((1,), (1,)), ((), ())),
                          preferred_element_type=jnp.float32)
    out = out + b_ref[...]
    m = jnp.max(out, axis=0, keepdims=True)
    e = jnp.exp(out - m)
    s = jnp.sum(e, axis=0, keepdims=True)
    o_ref[...] = out - m - jnp.log(s)


def _tc_head(t0p, t1p, iw, fc_wp, fc_b2):
    nblk = _BATCH // _TC_BLK
    return pl.pallas_call(
        _tc_head_body,
        grid=(nblk,),
        in_specs=[
            pl.BlockSpec((_TC_BLK, _DIMP), lambda i: (i, 0)),
            pl.BlockSpec((_TC_BLK, _DIMP), lambda i: (i, 0)),
            pl.BlockSpec((_TC_BLK, 2), lambda i: (i, 0)),
            pl.BlockSpec((_OUT, _DIMP), lambda i: (0, 0)),
            pl.BlockSpec((_OUT, 1), lambda i: (0, 0)),
        ],
        out_specs=pl.BlockSpec((_OUT, _TC_BLK), lambda i: (0, i)),
        out_shape=jax.ShapeDtypeStruct((_OUT, _BATCH), jnp.float32),
    )(t0p, t1p, iw, fc_wp, fc_b2)


def kernel(x, table, importance, fc_w, fc_b):
    table_p = _pad_table(table.T)
    table_p, imp_t = lax.optimization_barrier((table_p, importance.T))
    imp0 = imp_t[0]
    imp1 = imp_t[1]
    fc_wpt = jnp.pad(fc_w.T, ((0, 0), (0, _DIMP - _DIM)))
    t0p, t1p = _sc_gather_tables(x, table_p)
    imp0, imp1 = lax.optimization_barrier((imp0, imp1, t0p))[:2]
    w0, w1 = _sc_gather_imp(x, imp0, imp1)
    iw = jnp.stack([w0, w1], axis=1)
    out_t = _tc_head(t0p, t1p, iw, fc_wpt, fc_b.reshape(_OUT, 1))
    return out_t.T


# speedup vs baseline: 1.4269x; 1.0653x over previous
import functools

import jax
import jax.numpy as jnp
from jax import lax
from jax.experimental import pallas as pl
from jax.experimental.pallas import tpu as pltpu
from jax.experimental.pallas import tpu_sc as plsc

_NUM_BUCKETS = 100000
_DIM = 100
_DIMP = 128
_PRIME0, _PRIME1 = 31, 1009
_OFF0, _OFF1 = 7, 433
_BATCH = 16384
_OUT = 300

_NC = 2
_NS = 16
_NW = _NC * _NS
_BPW = _BATCH // _NW
_SUB = 256
_LANES = 16

_sc_mesh = plsc.VectorSubcoreMesh(core_axis_name="c", subcore_axis_name="s")


def _sc_gather_tables(x, table_p):
    @functools.partial(
        pl.kernel,
        mesh=_sc_mesh,
        out_type=(
            jax.ShapeDtypeStruct((_BATCH, _DIMP), jnp.float32),
            jax.ShapeDtypeStruct((_BATCH, _DIMP), jnp.float32),
        ),
        scratch_types=[
            pltpu.VMEM((_BPW,), jnp.int32),
            pltpu.VMEM((_SUB,), jnp.int32),
            pltpu.VMEM((_SUB,), jnp.int32),
            pltpu.VMEM((_SUB, _DIMP), jnp.float32),
            pltpu.VMEM((_SUB, _DIMP), jnp.float32),
            pltpu.SemaphoreType.DMA,
        ],
    )
    def tab_kernel(x_hbm, table_hbm, t0_hbm, t1_hbm,
                   x_v, h0_v, h1_v, t0_v, t1_v, sem):
        wid = lax.axis_index("s") * _NC + lax.axis_index("c")
        base = wid * _BPW
        pltpu.sync_copy(x_hbm.at[pl.ds(base, _BPW)], x_v)

        for s in range(_BPW // _SUB):
            def hash_body(i, carry, s=s):
                xx = x_v[pl.ds(s * _SUB + i * _LANES, _LANES)]
                sl = pl.ds(i * _LANES, _LANES)
                h0_v[sl] = (xx * _PRIME0 + _OFF0) % _NUM_BUCKETS
                h1_v[sl] = (xx * _PRIME1 + _OFF1) % _NUM_BUCKETS
                return carry

            lax.fori_loop(0, _SUB // _LANES, hash_body, 0)

            cp0 = pltpu.async_copy(table_hbm.at[h0_v], t0_v, sem)
            cp1 = pltpu.async_copy(table_hbm.at[h1_v], t1_v, sem)
            cp0.wait()
            cp1.wait()

            out_sl = pl.ds(base + s * _SUB, _SUB)
            pltpu.sync_copy(t0_v, t0_hbm.at[out_sl])
            pltpu.sync_copy(t1_v, t1_hbm.at[out_sl])

    return tab_kernel(x, table_p)


def _sc_gather_imp(x, imp0, imp1):
    @functools.partial(
        pl.kernel,
        mesh=_sc_mesh,
        out_type=(
            jax.ShapeDtypeStruct((_BATCH,), jnp.float32),
            jax.ShapeDtypeStruct((_BATCH,), jnp.float32),
        ),
        scratch_types=[
            pltpu.VMEM((_BPW,), jnp.int32),
            pltpu.VMEM((_BPW,), jnp.float32),
            pltpu.VMEM((_BPW,), jnp.float32),
            pltpu.SemaphoreType.DMA,
        ],
    )
    def imp_kernel(x_hbm, imp0_hbm, imp1_hbm, w0_hbm, w1_hbm,
                   x_v, i0_v, i1_v, sem):
        wid = lax.axis_index("s") * _NC + lax.axis_index("c")
        base = wid * _BPW
        sl = pl.ds(base, _BPW)
        pltpu.sync_copy(x_hbm.at[sl], x_v)
        cp0 = pltpu.async_copy(imp0_hbm.at[x_v], i0_v, sem)
        cp1 = pltpu.async_copy(imp1_hbm.at[x_v], i1_v, sem)
        cp0.wait()
        cp1.wait()
        pltpu.sync_copy(i0_v, w0_hbm.at[sl])
        pltpu.sync_copy(i1_v, w1_hbm.at[sl])

    return imp_kernel(x, imp0, imp1)


_PAD_BLK = 8192


def _pad_body(t_ref, o_ref):
    o_ref[:, : _DIM] = t_ref[...].T
    o_ref[:, _DIM:] = jnp.zeros((_PAD_BLK, _DIMP - _DIM), jnp.float32)


def _pad_table(table_t):
    nblk = (_NUM_BUCKETS + _PAD_BLK - 1) // _PAD_BLK
    return pl.pallas_call(
        _pad_body,
        grid=(nblk,),
        in_specs=[pl.BlockSpec((_DIM, _PAD_BLK), lambda i: (0, i))],
        out_specs=pl.BlockSpec((_PAD_BLK, _DIMP), lambda i: (i, 0)),
        out_shape=jax.ShapeDtypeStruct((_NUM_BUCKETS, _DIMP), jnp.float32),
    )(table_t)


_TC_BLK = 4096


def _tc_head_body(t0_ref, t1_ref, w0_ref, w1_ref, w_ref, b_ref, o_ref):
    t0 = t0_ref[...]
    t1 = t1_ref[...]
    w0 = w0_ref[...].reshape(_TC_BLK, 1)
    w1 = w1_ref[...].reshape(_TC_BLK, 1)
    emb = w0 * t0 + w1 * t1
    out = lax.dot_general(w_ref[...], emb, (((1,), (1,)), ((), ())),
                          preferred_element_type=jnp.float32)
    out = out + b_ref[...]
    m = jnp.max(out, axis=0, keepdims=True)
    e = jnp.exp(out - m)
    s = jnp.sum(e, axis=0, keepdims=True)
    o_ref[...] = out - m - jnp.log(s)


def _tc_head(t0p, t1p, w0, w1, fc_wp, fc_b2):
    nblk = _BATCH // _TC_BLK
    return pl.pallas_call(
        _tc_head_body,
        grid=(nblk,),
        in_specs=[
            pl.BlockSpec((_TC_BLK, _DIMP), lambda i: (i, 0)),
            pl.BlockSpec((_TC_BLK, _DIMP), lambda i: (i, 0)),
            pl.BlockSpec((_TC_BLK,), lambda i: (i,)),
            pl.BlockSpec((_TC_BLK,), lambda i: (i,)),
            pl.BlockSpec((_OUT, _DIMP), lambda i: (0, 0)),
            pl.BlockSpec((_OUT, 1), lambda i: (0, 0)),
        ],
        out_specs=pl.BlockSpec((_OUT, _TC_BLK), lambda i: (0, i)),
        out_shape=jax.ShapeDtypeStruct((_OUT, _BATCH), jnp.float32),
    )(t0p, t1p, w0, w1, fc_wp, fc_b2)


def kernel(x, table, importance, fc_w, fc_b):
    table_p = _pad_table(table.T)
    table_p, imp_t = lax.optimization_barrier((table_p, importance.T))
    imp0 = imp_t[0]
    imp1 = imp_t[1]
    fc_wpt = jnp.pad(fc_w.T, ((0, 0), (0, _DIMP - _DIM)))
    t0p, t1p = _sc_gather_tables(x, table_p)
    imp0, imp1 = lax.optimization_barrier((imp0, imp1, t0p))[:2]
    w0, w1 = _sc_gather_imp(x, imp0, imp1)
    out_t = _tc_head(t0p, t1p, w0, w1, fc_wpt, fc_b.reshape(_OUT, 1))
    return out_t.T
